# baseline (speedup 1.0000x reference)
import functools
import math

import jax
import jax.numpy as jnp
from jax.experimental import pallas as pl
from jax.experimental.pallas import tpu as pltpu

_NUM_HEADS = 12
_EPS = 1e-5
_CORES = 2
_HCH = 3
_FCH = 4


def _layernorm_f32(x, g, b, inv_d):
    s1 = jnp.sum(x, axis=-1, keepdims=True)
    s2 = jnp.sum(x * x, axis=-1, keepdims=True)
    mean = s1 * inv_d
    var = s2 * inv_d - mean * mean
    inv_std = jax.lax.rsqrt(var + _EPS)
    scale = g * inv_std
    shift = b - mean * scale
    return x * scale + shift


def _attn_kernel(x_ref, wqkv_ref, bqkv_ref, o_ref, *, seq, d_model, bpb):
    hd = d_model // _NUM_HEADS
    hpc = _NUM_HEADS // _HCH
    sm_scale = 1.0 / math.sqrt(hd)

    xb = x_ref[...].astype(jnp.bfloat16)
    qkv = jnp.dot(xb, wqkv_ref[...].astype(jnp.bfloat16),
                  preferred_element_type=jnp.float32)
    qkv = qkv + bqkv_ref[...]

    chains = [(bi, h) for bi in range(bpb) for h in range(hpc)]
    scores = []
    vs = []
    for bi, h in chains:
        base = h * 3 * hd
        r0 = bi * seq
        qh = qkv[r0:r0 + seq, base:base + hd].astype(jnp.bfloat16)
        kh = qkv[r0:r0 + seq, base + hd:base + 2 * hd].astype(jnp.bfloat16)
        vs.append(qkv[r0:r0 + seq,
                      base + 2 * hd:base + 3 * hd].astype(jnp.bfloat16))
        scores.append(jax.lax.dot_general(
            qh, kh, (((1,), (1,)), ((), ())),
            preferred_element_type=jnp.float32))
    c2 = sm_scale * 1.4426950408889634
    probs = []
    for s in scores:
        s = s - jnp.max(s, axis=-1, keepdims=True)
        p = jnp.exp2(s * c2)
        p = p / jnp.sum(p, axis=-1, keepdims=True)
        probs.append(p.astype(jnp.bfloat16))
    for (bi, h), p, v in zip(chains, probs, vs):
        oh = jnp.dot(p, v, preferred_element_type=jnp.float32)
        o_ref[bi, h * seq:(h + 1) * seq, :] = oh.astype(jnp.bfloat16)


def _ffn_kernel(v_ref, x_ref, wo_ref, bo_ref, w1_ref, b1_ref,
                w2_ref, b2_ref, g1_ref, bt1_ref, g2_ref, bt2_ref, o_ref,
                h1_s, acc_s, *, d_model):
    inv_d = 1.0 / d_model
    n = pl.program_id(1)
    rt = v_ref.shape[0]
    hrt = rt // 2
    halves = (0, hrt)

    @pl.when(n == 0)
    def _first():
        wo_b = wo_ref[...].astype(jnp.bfloat16)
        attn = [jnp.dot(v_ref[r:r + hrt, :], wo_b,
                        preferred_element_type=jnp.float32) + bo_ref[...]
                for r in halves]
        for a, r in zip(attn, halves):
            h1_s[r:r + hrt, :] = _layernorm_f32(
                a + x_ref[r:r + hrt, :], g1_ref[...], bt1_ref[...], inv_d)

    w1_b = w1_ref[...].astype(jnp.bfloat16)
    w2_b = w2_ref[...].astype(jnp.bfloat16)
    ff = [jnp.maximum(jnp.dot(h1_s[r:r + hrt, :].astype(jnp.bfloat16), w1_b,
                              preferred_element_type=jnp.float32)
                      + b1_ref[...], 0.0)
          for r in halves]
    part = [jnp.dot(f.astype(jnp.bfloat16), w2_b,
                    preferred_element_type=jnp.float32) for f in ff]

    @pl.when(n == 0)
    def _acc_init():
        for p, r in zip(part, halves):
            acc_s[r:r + hrt, :] = p

    @pl.when(jnp.logical_and(n > 0, n < _FCH - 1))
    def _acc_add():
        for p, r in zip(part, halves):
            acc_s[r:r + hrt, :] += p

    @pl.when(n == _FCH - 1)
    def _final():
        for p, r in zip(part, halves):
            ff2 = acc_s[r:r + hrt, :] + p + b2_ref[...]
            o_ref[r:r + hrt, :] = _layernorm_f32(
                ff2 + h1_s[r:r + hrt, :], g2_ref[...], bt2_ref[...], inv_d)


def kernel(x, w_qkv, b_qkv, w_o, b_o, w1, b1, w2, b2,
           gamma1, beta1, gamma2, beta2):
    b, s, d = x.shape
    dff = w1.shape[1]
    hd = d // _NUM_HEADS
    rows = b * s
    x2 = x.reshape(rows, d)

    def const(shape):
        return pl.BlockSpec(shape, lambda c, j: (0,) * len(shape))

    bpb = b // _CORES
    wcols = 3 * d // _HCH
    vals = pl.pallas_call(
        functools.partial(_attn_kernel, seq=s, d_model=d, bpb=bpb),
        out_shape=jax.ShapeDtypeStruct((b, _NUM_HEADS * s, hd), jnp.bfloat16),
        grid=(_CORES, _HCH),
        in_specs=[
            pl.BlockSpec((bpb * s, d), lambda c, j: (c, 0)),
            pl.BlockSpec((d, wcols), lambda c, j: (0, j)),
            pl.BlockSpec((1, wcols), lambda c, j: (0, j)),
        ],
        out_specs=pl.BlockSpec((bpb, (_NUM_HEADS // _HCH) * s, hd),
                               lambda c, j: (c, j, 0)),
        compiler_params=pltpu.CompilerParams(
            dimension_semantics=("parallel", "arbitrary"),
            vmem_limit_bytes=57 * 1024 * 1024,
        ),
    )(x2, w_qkv, b_qkv.reshape(1, 3 * d))

    vals2 = vals.reshape(rows, d)

    row_tile = rows // _CORES
    fcols = dff // _FCH
    out = pl.pallas_call(
        functools.partial(_ffn_kernel, d_model=d),
        out_shape=jax.ShapeDtypeStruct((rows, d), jnp.float32),
        grid=(_CORES, _FCH),
        in_specs=[
            pl.BlockSpec((row_tile, d), lambda c, j: (c, 0)),
            pl.BlockSpec((row_tile, d), lambda c, j: (c, 0)),
            const((d, d)),
            const((1, d)),
            pl.BlockSpec((d, fcols), lambda c, j: (0, j)),
            pl.BlockSpec((1, fcols), lambda c, j: (0, j)),
            pl.BlockSpec((fcols, d), lambda c, j: (j, 0)),
            const((1, d)),
            const((1, d)),
            const((1, d)),
            const((1, d)),
            const((1, d)),
        ],
        out_specs=pl.BlockSpec((row_tile, d), lambda c, j: (c, 0)),
        scratch_shapes=[
            pltpu.VMEM((row_tile, d), jnp.float32),
            pltpu.VMEM((row_tile, d), jnp.float32),
        ],
        compiler_params=pltpu.CompilerParams(
            dimension_semantics=("parallel", "arbitrary"),
            vmem_limit_bytes=57 * 1024 * 1024,
        ),
    )(vals2, x2, w_o, b_o.reshape(1, d),
      w1, b1.reshape(1, dff), w2, b2.reshape(1, d),
      gamma1.reshape(1, d), beta1.reshape(1, d),
      gamma2.reshape(1, d), beta2.reshape(1, d))
    return out.reshape(b, s, d)


# speedup vs baseline: 1.0139x; 1.0139x over previous
import functools
import math

import jax
import jax.numpy as jnp
from jax.experimental import pallas as pl
from jax.experimental.pallas import tpu as pltpu

_NUM_HEADS = 12
_EPS = 1e-5
_CORES = 2
_HCH = 2
_FCH = 2


def _layernorm_f32(x, g, b, inv_d):
    s1 = jnp.sum(x, axis=-1, keepdims=True)
    s2 = jnp.sum(x * x, axis=-1, keepdims=True)
    mean = s1 * inv_d
    var = s2 * inv_d - mean * mean
    inv_std = jax.lax.rsqrt(var + _EPS)
    scale = g * inv_std
    shift = b - mean * scale
    return x * scale + shift


def _attn_kernel(x_ref, wqkv_ref, bqkv_ref, o_ref, xb_s,
                 *, seq, d_model, bpb):
    hd = d_model // _NUM_HEADS
    hpc = _NUM_HEADS // _HCH
    sm_scale = 1.0 / math.sqrt(hd)

    @pl.when(pl.program_id(1) == 0)
    def _init():
        xb_s[...] = x_ref[...].astype(jnp.bfloat16)

    qkv = jnp.dot(xb_s[...], wqkv_ref[...].astype(jnp.bfloat16),
                  preferred_element_type=jnp.float32)
    qkv = qkv + bqkv_ref[...]

    chains = [(bi, h) for bi in range(bpb) for h in range(hpc)]
    scores = []
    vs = []
    for bi, h in chains:
        base = h * 3 * hd
        r0 = bi * seq
        qh = qkv[r0:r0 + seq, base:base + hd].astype(jnp.bfloat16)
        kh = qkv[r0:r0 + seq, base + hd:base + 2 * hd].astype(jnp.bfloat16)
        vs.append(qkv[r0:r0 + seq,
                      base + 2 * hd:base + 3 * hd].astype(jnp.bfloat16))
        scores.append(jax.lax.dot_general(
            qh, kh, (((1,), (1,)), ((), ())),
            preferred_element_type=jnp.float32))
    c2 = sm_scale * 1.4426950408889634
    probs = []
    for s in scores:
        s = s - jnp.max(s, axis=-1, keepdims=True)
        p = jnp.exp2(s * c2)
        p = p / jnp.sum(p, axis=-1, keepdims=True)
        probs.append(p.astype(jnp.bfloat16))
    for (bi, h), p, v in zip(chains, probs, vs):
        oh = jnp.dot(p, v, preferred_element_type=jnp.float32)
        o_ref[bi, h * seq:(h + 1) * seq, :] = oh.astype(jnp.bfloat16)


def _ffn_kernel(v_ref, x_ref, wo_ref, bo_ref, w1_ref, b1_ref,
                w2_ref, b2_ref, g1_ref, bt1_ref, g2_ref, bt2_ref, o_ref,
                h1_s, acc_s, *, d_model):
    inv_d = 1.0 / d_model
    n = pl.program_id(1)
    rt = v_ref.shape[0]
    hrt = rt // 2
    halves = (0, hrt)

    @pl.when(n == 0)
    def _first():
        wo_b = wo_ref[...].astype(jnp.bfloat16)
        attn = [jnp.dot(v_ref[r:r + hrt, :], wo_b,
                        preferred_element_type=jnp.float32) + bo_ref[...]
                for r in halves]
        for a, r in zip(attn, halves):
            h1_s[r:r + hrt, :] = _layernorm_f32(
                a + x_ref[r:r + hrt, :], g1_ref[...], bt1_ref[...], inv_d)

    w1_b = w1_ref[...].astype(jnp.bfloat16)
    w2_b = w2_ref[...].astype(jnp.bfloat16)
    ff = [jnp.maximum(jnp.dot(h1_s[r:r + hrt, :].astype(jnp.bfloat16), w1_b,
                              preferred_element_type=jnp.float32)
                      + b1_ref[...], 0.0)
          for r in halves]
    part = [jnp.dot(f.astype(jnp.bfloat16), w2_b,
                    preferred_element_type=jnp.float32) for f in ff]

    @pl.when(n == 0)
    def _acc_init():
        for p, r in zip(part, halves):
            acc_s[r:r + hrt, :] = p

    @pl.when(n == _FCH - 1)
    def _final():
        for p, r in zip(part, halves):
            ff2 = acc_s[r:r + hrt, :] + p + b2_ref[...]
            o_ref[r:r + hrt, :] = _layernorm_f32(
                ff2 + h1_s[r:r + hrt, :], g2_ref[...], bt2_ref[...], inv_d)


def kernel(x, w_qkv, b_qkv, w_o, b_o, w1, b1, w2, b2,
           gamma1, beta1, gamma2, beta2):
    b, s, d = x.shape
    dff = w1.shape[1]
    hd = d // _NUM_HEADS
    rows = b * s
    x2 = x.reshape(rows, d)

    def const(shape):
        return pl.BlockSpec(shape, lambda c, j: (0,) * len(shape))

    bpb = b // _CORES
    wcols = 3 * d // _HCH
    vals = pl.pallas_call(
        functools.partial(_attn_kernel, seq=s, d_model=d, bpb=bpb),
        out_shape=jax.ShapeDtypeStruct((b, _NUM_HEADS * s, hd), jnp.bfloat16),
        grid=(_CORES, _HCH),
        in_specs=[
            pl.BlockSpec((bpb * s, d), lambda c, j: (c, 0)),
            pl.BlockSpec((d, wcols), lambda c, j: (0, j)),
            pl.BlockSpec((1, wcols), lambda c, j: (0, j)),
        ],
        out_specs=pl.BlockSpec((bpb, (_NUM_HEADS // _HCH) * s, hd),
                               lambda c, j: (c, j, 0)),
        scratch_shapes=[pltpu.VMEM((bpb * s, d), jnp.bfloat16)],
        compiler_params=pltpu.CompilerParams(
            dimension_semantics=("parallel", "arbitrary"),
            vmem_limit_bytes=57 * 1024 * 1024,
        ),
    )(x2, w_qkv, b_qkv.reshape(1, 3 * d))

    vals2 = vals.reshape(rows, d)

    row_tile = rows // _CORES
    fcols = dff // _FCH
    out = pl.pallas_call(
        functools.partial(_ffn_kernel, d_model=d),
        out_shape=jax.ShapeDtypeStruct((rows, d), jnp.float32),
        grid=(_CORES, _FCH),
        in_specs=[
            pl.BlockSpec((row_tile, d), lambda c, j: (c, 0)),
            pl.BlockSpec((row_tile, d), lambda c, j: (c, 0)),
            const((d, d)),
            const((1, d)),
            pl.BlockSpec((d, fcols), lambda c, j: (0, j)),
            pl.BlockSpec((1, fcols), lambda c, j: (0, j)),
            pl.BlockSpec((fcols, d), lambda c, j: (j, 0)),
            const((1, d)),
            const((1, d)),
            const((1, d)),
            const((1, d)),
            const((1, d)),
        ],
        out_specs=pl.BlockSpec((row_tile, d), lambda c, j: (c, 0)),
        scratch_shapes=[
            pltpu.VMEM((row_tile, d), jnp.float32),
            pltpu.VMEM((row_tile, d), jnp.float32),
        ],
        compiler_params=pltpu.CompilerParams(
            dimension_semantics=("parallel", "arbitrary"),
            vmem_limit_bytes=57 * 1024 * 1024,
        ),
    )(vals2, x2, w_o, b_o.reshape(1, d),
      w1, b1.reshape(1, dff), w2, b2.reshape(1, d),
      gamma1.reshape(1, d), beta1.reshape(1, d),
      gamma2.reshape(1, d), beta2.reshape(1, d))
    return out.reshape(b, s, d)


# speedup vs baseline: 1.0818x; 1.0670x over previous
import functools
import math

import jax
import jax.numpy as jnp
from jax.experimental import pallas as pl
from jax.experimental.pallas import tpu as pltpu

_NUM_HEADS = 12
_EPS = 1e-5
_CORES = 2


def _layernorm_f32(x, g, b, inv_d):
    s1 = jnp.sum(x, axis=-1, keepdims=True)
    s2 = jnp.sum(x * x, axis=-1, keepdims=True)
    mean = s1 * inv_d
    var = s2 * inv_d - mean * mean
    inv_std = jax.lax.rsqrt(var + _EPS)
    scale = g * inv_std
    shift = b - mean * scale
    return x * scale + shift


def _attn_kernel(x_ref, wqkv_ref, bqkv_ref, o_ref, wqkv_s,
                 *, seq, d_model, bpb):
    hd = d_model // _NUM_HEADS
    sm_scale = 1.0 / math.sqrt(hd)

    @pl.when(pl.program_id(1) == 0)
    def _init():
        wqkv_s[...] = wqkv_ref[...].astype(jnp.bfloat16)

    xb = x_ref[...].astype(jnp.bfloat16)
    qkv = jnp.dot(xb, wqkv_s[...], preferred_element_type=jnp.float32)
    qkv = qkv + bqkv_ref[...]

    chains = [(bi, h) for bi in range(bpb) for h in range(_NUM_HEADS)]
    scores = []
    vs = []
    for bi, h in chains:
        base = h * 3 * hd
        r0 = bi * seq
        qh = qkv[r0:r0 + seq, base:base + hd].astype(jnp.bfloat16)
        kh = qkv[r0:r0 + seq, base + hd:base + 2 * hd].astype(jnp.bfloat16)
        vs.append(qkv[r0:r0 + seq,
                      base + 2 * hd:base + 3 * hd].astype(jnp.bfloat16))
        scores.append(jax.lax.dot_general(
            qh, kh, (((1,), (1,)), ((), ())),
            preferred_element_type=jnp.float32))
    c2 = sm_scale * 1.4426950408889634
    probs = []
    for s in scores:
        s = s - jnp.max(s, axis=-1, keepdims=True)
        p = jnp.exp2(s * c2)
        p = p / jnp.sum(p, axis=-1, keepdims=True)
        probs.append(p.astype(jnp.bfloat16))
    for (bi, h), p, v in zip(chains, probs, vs):
        oh = jnp.dot(p, v, preferred_element_type=jnp.float32)
        o_ref[bi, h * seq:(h + 1) * seq, :] = oh.astype(jnp.bfloat16)


def _ffn_kernel(v_ref, x_ref, wo_ref, bo_ref, w1_ref, b1_ref,
                w2_ref, b2_ref, g1_ref, bt1_ref, g2_ref, bt2_ref, o_ref,
                wo_s, w1_s, w2_s, *, d_model):
    inv_d = 1.0 / d_model

    @pl.when(pl.program_id(1) == 0)
    def _init():
        wo_s[...] = wo_ref[...].astype(jnp.bfloat16)
        w1_s[...] = w1_ref[...].astype(jnp.bfloat16)
        w2_s[...] = w2_ref[...].astype(jnp.bfloat16)

    rt = v_ref.shape[0]
    halves = range(0, rt, rt // 2)
    attn = [jnp.dot(v_ref[r:r + rt // 2, :], wo_s[...],
                    preferred_element_type=jnp.float32) + bo_ref[...]
            for r in halves]
    h1 = [_layernorm_f32(a + x_ref[r:r + rt // 2, :],
                         g1_ref[...], bt1_ref[...], inv_d)
          for a, r in zip(attn, halves)]
    ff = [jnp.maximum(jnp.dot(h.astype(jnp.bfloat16), w1_s[...],
                              preferred_element_type=jnp.float32)
                      + b1_ref[...], 0.0)
          for h in h1]
    ff2 = [jnp.dot(f.astype(jnp.bfloat16), w2_s[...],
                   preferred_element_type=jnp.float32) + b2_ref[...]
           for f in ff]
    for f2, h, r in zip(ff2, h1, halves):
        o_ref[r:r + rt // 2, :] = _layernorm_f32(
            f2 + h, g2_ref[...], bt2_ref[...], inv_d)


def kernel(x, w_qkv, b_qkv, w_o, b_o, w1, b1, w2, b2,
           gamma1, beta1, gamma2, beta2):
    b, s, d = x.shape
    dff = w1.shape[1]
    hd = d // _NUM_HEADS
    rows = b * s
    x2 = x.reshape(rows, d)

    def const(shape):
        return pl.BlockSpec(shape, lambda c, j: (0,) * len(shape))

    bpb = 8 if b % (8 * _CORES) == 0 else 1
    asteps = b // (_CORES * bpb)
    vals = pl.pallas_call(
        functools.partial(_attn_kernel, seq=s, d_model=d, bpb=bpb),
        out_shape=jax.ShapeDtypeStruct((b, _NUM_HEADS * s, hd), jnp.bfloat16),
        grid=(_CORES, asteps),
        in_specs=[
            pl.BlockSpec((bpb * s, d),
                         lambda c, j: (c * asteps + j, 0)),
            const((d, 3 * d)),
            const((1, 3 * d)),
        ],
        out_specs=pl.BlockSpec((bpb, _NUM_HEADS * s, hd),
                               lambda c, j: (c * asteps + j, 0, 0)),
        scratch_shapes=[pltpu.VMEM((d, 3 * d), jnp.bfloat16)],
        compiler_params=pltpu.CompilerParams(
            dimension_semantics=("parallel", "arbitrary"),
            vmem_limit_bytes=57 * 1024 * 1024,
        ),
    )(x2, w_qkv, b_qkv.reshape(1, 3 * d))

    vals2 = vals.reshape(rows, d)

    row_tile = 1024 if rows % (1024 * _CORES) == 0 else rows // _CORES
    fsteps = rows // (row_tile * _CORES)
    out = pl.pallas_call(
        functools.partial(_ffn_kernel, d_model=d),
        out_shape=jax.ShapeDtypeStruct((rows, d), jnp.float32),
        grid=(_CORES, fsteps),
        in_specs=[
            pl.BlockSpec((row_tile, d), lambda c, j: (c * fsteps + j, 0)),
            pl.BlockSpec((row_tile, d), lambda c, j: (c * fsteps + j, 0)),
            const((d, d)),
            const((1, d)),
            const((d, dff)),
            const((1, dff)),
            const((dff, d)),
            const((1, d)),
            const((1, d)),
            const((1, d)),
            const((1, d)),
            const((1, d)),
        ],
        out_specs=pl.BlockSpec((row_tile, d), lambda c, j: (c * fsteps + j, 0)),
        scratch_shapes=[
            pltpu.VMEM((d, d), jnp.bfloat16),
            pltpu.VMEM((d, dff), jnp.bfloat16),
            pltpu.VMEM((dff, d), jnp.bfloat16),
        ],
        compiler_params=pltpu.CompilerParams(
            dimension_semantics=("parallel", "arbitrary"),
            vmem_limit_bytes=57 * 1024 * 1024,
        ),
    )(vals2, x2, w_o, b_o.reshape(1, d),
      w1, b1.reshape(1, dff), w2, b2.reshape(1, d),
      gamma1.reshape(1, d), beta1.reshape(1, d),
      gamma2.reshape(1, d), beta2.reshape(1, d))
    return out.reshape(b, s, d)
